# baseline (speedup 1.0000x reference)
import functools

import jax
import jax.numpy as jnp
from jax import lax
from jax.experimental import pallas as pl
from jax.experimental.pallas import tpu as pltpu
from jax.experimental.pallas import tpu_sc as plsc

N = 10000
E = 320000
D = 128
DH = D // 2

NC = 2
NS = 16
K = 128


G = 4
PER_TILE = -(-E // (NS * K * G)) * K * G
EP = PER_TILE * NS
CHUNKS = PER_TILE // K
HC = CHUNKS // 2

ACC = 10240
RPT = ACC // NS
ZC = RPT // K


def _sc_agg_body(src4_hbm, dst3_hbm, xcat_hbm, z2_hbm, z1_hbm, ones_hbm,
                 sums_out, cnt_out,
                 src_all, dst_all, rowsg, ones_v, cbuf_v,
                 acc_sh, cnt_sh, sem0):
    cid = lax.axis_index("c")
    sid = lax.axis_index("s")
    row0 = sid * RPT

    pltpu.sync_copy(src4_hbm.at[cid, sid], src_all)
    pltpu.sync_copy(dst3_hbm.at[sid], dst_all)
    pltpu.sync_copy(z2_hbm, rowsg.at[0])
    for z in range(ZC):
        pltpu.sync_copy(rowsg.at[0], acc_sh.at[pl.ds(row0 + z * K, K)])
    pltpu.sync_copy(z1_hbm, cbuf_v)
    pltpu.sync_copy(cbuf_v, cnt_sh.at[pl.ds(row0, RPT)])
    pltpu.sync_copy(ones_hbm, ones_v)
    plsc.subcore_barrier()

    def count_mine(j):
        return lax.select(cid == 0, j < HC, j >= HC)

    def body(j, carry):
        pltpu.async_copy(xcat_hbm.at[src_all.at[j]], rowsg.at[0], sem0).wait()
        pltpu.sync_copy(rowsg.at[0], acc_sh.at[dst_all.at[j]], add=True)

        @pl.when(count_mine(j))
        def _():
            pltpu.sync_copy(ones_v, cnt_sh.at[dst_all.at[j]], add=True)

        return carry

    lax.fori_loop(0, CHUNKS, body, 0)
    plsc.subcore_barrier()

    for z in range(ZC):
        pltpu.sync_copy(acc_sh.at[pl.ds(row0 + z * K, K)], rowsg.at[0])
        pltpu.sync_copy(rowsg.at[0], sums_out.at[cid, pl.ds(row0 + z * K, K)])
    pltpu.sync_copy(cnt_sh.at[pl.ds(row0, RPT)], cbuf_v)
    pltpu.sync_copy(cbuf_v, cnt_out.at[cid, pl.ds(row0, RPT)])


_sc_agg = functools.partial(
    pl.kernel,
    out_type=(
        jax.ShapeDtypeStruct((NC, ACC, DH), jnp.float32),
        jax.ShapeDtypeStruct((NC, ACC), jnp.float32),
    ),
    mesh=plsc.VectorSubcoreMesh(core_axis_name="c", subcore_axis_name="s"),
    compiler_params=pltpu.CompilerParams(use_tc_tiling_on_sc=False),
    scratch_types=(
        pltpu.VMEM((CHUNKS, K), jnp.int32),
        pltpu.VMEM((CHUNKS, K), jnp.int32),
        pltpu.VMEM((G, K, DH), jnp.float32),
        pltpu.VMEM((K,), jnp.float32),
        pltpu.VMEM((RPT,), jnp.float32),
        pltpu.VMEM_SHARED((ACC, DH), jnp.float32),
        pltpu.VMEM_SHARED((ACC,), jnp.float32),
        pltpu.SemaphoreType.DMA,
    ),
)(_sc_agg_body)


BR = 512


def _tc_dense_body(x_ref, s_ref, c_ref, wl_ref, wr_ref, w1_ref, bl_ref, b1_ref,
                   o_ref):
    c = c_ref[:, 0:1] + c_ref[:, 1:2]
    inv = 1.0 / jnp.maximum(c, 1.0)
    m0 = s_ref[0] * inv
    m1 = s_ref[1] * inv
    h = jnp.dot(m0, wl_ref[0:DH, :], preferred_element_type=jnp.float32)
    h += jnp.dot(m1, wl_ref[DH:D, :], preferred_element_type=jnp.float32)
    h += jnp.dot(x_ref[...], wr_ref[...], preferred_element_type=jnp.float32)
    h = jnp.maximum(h + bl_ref[...], 0.0)
    o_ref[...] = (jnp.dot(h, w1_ref[...], preferred_element_type=jnp.float32)
                  + b1_ref[...])


def kernel(x, edge_index, Wl, bl, Wr, W1, b1):
    src = edge_index[0]
    dst = edge_index[1]
    pad = EP - E
    src_p = jnp.concatenate([src, jnp.zeros((pad,), jnp.int32)])
    dst_p = jnp.concatenate([dst, jnp.full((pad,), N, jnp.int32)])
    xcat = jnp.concatenate([x[:, :DH], x[:, DH:]], axis=0)
    src4 = jnp.stack([src_p, src_p + N]).reshape(NC, NS, CHUNKS, K)
    dst3 = dst_p.reshape(NS, CHUNKS, K)
    z2 = jnp.zeros((K, DH), jnp.float32)
    z1 = jnp.zeros((RPT,), jnp.float32)
    ones = jnp.ones((K,), jnp.float32)

    sums, cnts = _sc_agg(src4, dst3, xcat, z2, z1, ones)

    grid = ACC // BR
    out = pl.pallas_call(
        _tc_dense_body,
        grid=(grid,),
        in_specs=[
            pl.BlockSpec((BR, D), lambda i: (i, 0)),
            pl.BlockSpec((NC, BR, DH), lambda i: (0, i, 0)),
            pl.BlockSpec((BR, NC), lambda i: (i, 0)),
            pl.BlockSpec((D, D), lambda i: (0, 0)),
            pl.BlockSpec((D, D), lambda i: (0, 0)),
            pl.BlockSpec((D, D), lambda i: (0, 0)),
            pl.BlockSpec((1, D), lambda i: (0, 0)),
            pl.BlockSpec((1, D), lambda i: (0, 0)),
        ],
        out_specs=pl.BlockSpec((BR, D), lambda i: (i, 0)),
        out_shape=jax.ShapeDtypeStruct((N, D), jnp.float32),
    )(x, sums, cnts.T, Wl.T, Wr.T, W1.T, bl[None, :], b1[None, :])
    return out


# speedup vs baseline: 1.4279x; 1.4279x over previous
import functools

import jax
import jax.numpy as jnp
from jax import lax
from jax.experimental import pallas as pl
from jax.experimental.pallas import tpu as pltpu
from jax.experimental.pallas import tpu_sc as plsc

N = 10000
E = 320000
D = 128
DH = D // 2

NC = 2
NS = 16
K = 128


G = 2
PER_TILE = -(-E // (NS * K * G)) * K * G
EP = PER_TILE * NS
CHUNKS = PER_TILE // K
HC = CHUNKS // 2
CW = 16

ACC = 10240
RPT = ACC // NS
ZC = RPT // K


def _sc_agg_body(src4_hbm, dst3_hbm, xcat_hbm, z2_hbm, z1_hbm, ones_hbm,
                 sums_out, cnt_out,
                 src_all, dst_all, rowsg, ones_v, cbuf_v, acc_sh, cnt_sh,
                 sem0):
    cid = lax.axis_index("c")
    sid = lax.axis_index("s")
    row0 = sid * RPT

    pltpu.sync_copy(src4_hbm.at[cid, sid], src_all)
    pltpu.sync_copy(dst3_hbm.at[sid], dst_all)
    pltpu.sync_copy(z2_hbm, rowsg.at[0])
    for z in range(ZC):
        pltpu.sync_copy(rowsg.at[0], acc_sh.at[pl.ds(row0 + z * K, K)])
    pltpu.sync_copy(z1_hbm, cbuf_v)
    pltpu.sync_copy(cbuf_v, cnt_sh.at[pl.ds(row0, RPT)])
    pltpu.sync_copy(ones_hbm, ones_v)
    plsc.subcore_barrier()

    def count_mine(j):
        return lax.select(cid == 0, j < HC, j >= HC)

    def body(i, carry):
        j0 = G * i
        descs = [
            pltpu.async_copy(xcat_hbm.at[src_all.at[j0 + b]], rowsg.at[b],
                             sem0)
            for b in range(G)
        ]
        for b, d in enumerate(descs):
            d.wait()
            pltpu.sync_copy(rowsg.at[b], acc_sh.at[dst_all.at[j0 + b]],
                            add=True)

            @pl.when(count_mine(j0 + b))
            def _():
                pltpu.sync_copy(ones_v, cnt_sh.at[dst_all.at[j0 + b]],
                                add=True)

        return carry

    lax.fori_loop(0, CHUNKS // G, body, 0)
    plsc.subcore_barrier()

    for z in range(ZC):
        pltpu.sync_copy(acc_sh.at[pl.ds(row0 + z * K, K)], rowsg.at[0])
        pltpu.sync_copy(rowsg.at[0], sums_out.at[cid, pl.ds(row0 + z * K, K)])

    pltpu.sync_copy(cnt_sh.at[pl.ds(row0, RPT)], cbuf_v)
    pltpu.sync_copy(cbuf_v, cnt_out.at[cid, pl.ds(row0, RPT)])


_sc_agg = functools.partial(
    pl.kernel,
    out_type=(
        jax.ShapeDtypeStruct((NC, ACC, DH), jnp.float32),
        jax.ShapeDtypeStruct((NC, ACC, CW), jnp.float32),
    ),
    mesh=plsc.VectorSubcoreMesh(core_axis_name="c", subcore_axis_name="s"),
    compiler_params=pltpu.CompilerParams(use_tc_tiling_on_sc=False),
    scratch_types=(
        pltpu.VMEM((CHUNKS, K), jnp.int32),
        pltpu.VMEM((CHUNKS, K), jnp.int32),
        pltpu.VMEM((G, K, DH), jnp.float32),
        pltpu.VMEM((K, CW), jnp.float32),
        pltpu.VMEM((RPT, CW), jnp.float32),
        pltpu.VMEM_SHARED((ACC, DH), jnp.float32),
        pltpu.VMEM_SHARED((ACC, CW), jnp.float32),
        pltpu.SemaphoreType.DMA,
    ),
)(_sc_agg_body)


BR = 512


def _tc_dense_body(x_ref, s_ref, c_ref, wl_ref, wr_ref, w1_ref, bl_ref, b1_ref,
                   o_ref):
    c = c_ref[0, :, 0:1] + c_ref[1, :, 0:1]
    inv = 1.0 / jnp.maximum(c, 1.0)
    m0 = s_ref[0] * inv
    m1 = s_ref[1] * inv
    h = jnp.dot(m0, wl_ref[0:DH, :], preferred_element_type=jnp.float32)
    h += jnp.dot(m1, wl_ref[DH:D, :], preferred_element_type=jnp.float32)
    h += jnp.dot(x_ref[...], wr_ref[...], preferred_element_type=jnp.float32)
    h = jnp.maximum(h + bl_ref[...], 0.0)
    o_ref[...] = (jnp.dot(h, w1_ref[...], preferred_element_type=jnp.float32)
                  + b1_ref[...])


def kernel(x, edge_index, Wl, bl, Wr, W1, b1):
    src = edge_index[0]
    dst = edge_index[1]
    pad = EP - E
    src_p = jnp.concatenate([src, jnp.zeros((pad,), jnp.int32)])
    dst_p = jnp.concatenate([dst, jnp.full((pad,), N, jnp.int32)])
    xcat = jnp.concatenate([x[:, :DH], x[:, DH:]], axis=0)
    src4 = jnp.stack([src_p, src_p + N]).reshape(NC, NS, CHUNKS, K)
    dst3 = dst_p.reshape(NS, CHUNKS, K)
    z2 = jnp.zeros((K, DH), jnp.float32)
    z1 = jnp.zeros((RPT, CW), jnp.float32)
    ones = jnp.ones((K, CW), jnp.float32)

    sums, cnts = _sc_agg(src4, dst3, xcat, z2, z1, ones)

    grid = ACC // BR
    out = pl.pallas_call(
        _tc_dense_body,
        grid=(grid,),
        in_specs=[
            pl.BlockSpec((BR, D), lambda i: (i, 0)),
            pl.BlockSpec((NC, BR, DH), lambda i: (0, i, 0)),
            pl.BlockSpec((NC, BR, CW), lambda i: (0, i, 0)),
            pl.BlockSpec((D, D), lambda i: (0, 0)),
            pl.BlockSpec((D, D), lambda i: (0, 0)),
            pl.BlockSpec((D, D), lambda i: (0, 0)),
            pl.BlockSpec((1, D), lambda i: (0, 0)),
            pl.BlockSpec((1, D), lambda i: (0, 0)),
        ],
        out_specs=pl.BlockSpec((BR, D), lambda i: (i, 0)),
        out_shape=jax.ShapeDtypeStruct((N, D), jnp.float32),
    )(x, sums, cnts, Wl.T, Wr.T, W1.T, bl[None, :], b1[None, :])
    return out
